# baseline (speedup 1.0000x reference)
import jax
import jax.numpy as jnp
from jax.experimental import pallas as pl
from jax.experimental.pallas import tpu as pltpu

_N = 4096
_BF = jnp.bfloat16

_CP = getattr(pltpu, "CompilerParams", None) or getattr(pltpu, "TPUCompilerParams")


def _leaky(x):
    return jnp.where(x > 0, x, 0.2 * x)


def _dot(a, b):
    return jnp.dot(a, b, preferred_element_type=jnp.float32)


def _bdot(a, b):
    return jnp.dot(a.astype(_BF), b.astype(_BF), preferred_element_type=jnp.float32)


def _bdot_t(a, b, out_dtype=jnp.float32):
    return jax.lax.dot_general(
        a.astype(_BF), b.astype(_BF),
        dimension_numbers=(((1,), (1,)), ((), ())),
        preferred_element_type=out_dtype)


def _enc0_kernel(x1_ref, x2_ref, w1, b1, w2, b2, w3, b3, wz, bz, g1,
                 zae1_ref, zae2_ref, s1a_ref, s1b_ref):
    for x_ref, zae_ref, s1_ref in ((x1_ref, zae1_ref, s1a_ref),
                                   (x2_ref, zae2_ref, s1b_ref)):
        x = x_ref[...]
        h = _leaky(_bdot(x, w1[...]) + b1[...])
        h = _leaky(_bdot(h, w2[...]) + b2[...])
        h = _leaky(_bdot(h, w3[...]) + b3[...])
        zae_ref[...] = _dot(h, wz[...]) + bz[...]
        s1_ref[...] = jnp.tanh(_bdot(x, g1[...])).astype(_BF)


def _enc0(x1, x2, p):
    m = x1.shape[0]
    bm = 512
    ws = [p['ae_e1_W'], p['ae_e1_b'].reshape(1, -1),
          p['ae_e2_W'], p['ae_e2_b'].reshape(1, -1),
          p['ae_e3_W'], p['ae_e3_b'].reshape(1, -1),
          p['ae_z_W'], p['ae_z_b'].reshape(1, -1),
          p['g_e1_W']]
    xspec = pl.BlockSpec((bm, x1.shape[1]), lambda i: (i, 0))
    in_specs = [xspec, xspec]
    in_specs += [pl.BlockSpec(w.shape, lambda i: (0, 0)) for w in ws]
    lat = pl.BlockSpec((bm, 20), lambda i: (i, 0))
    s1 = pl.BlockSpec((bm, 128), lambda i: (i, 0))
    return pl.pallas_call(
        _enc0_kernel,
        grid=(m // bm,),
        in_specs=in_specs,
        out_specs=[lat, lat, s1, s1],
        out_shape=[jax.ShapeDtypeStruct((m, 20), jnp.float32),
                   jax.ShapeDtypeStruct((m, 20), jnp.float32),
                   jax.ShapeDtypeStruct((m, 128), _BF),
                   jax.ShapeDtypeStruct((m, 128), _BF)],
        compiler_params=_CP(dimension_semantics=("parallel",)),
    )(x1, x2, *ws)


def _adj_mm(adj, s, w_next=None, act=None, out_dtype=jnp.float32,
            emit_bf16=False, bm=1024):
    m, k = adj.shape
    f = s.shape[1]
    fo = f if w_next is None else w_next.shape[1]

    def kern(a_ref, s_ref, *rest):
        rest = list(rest)
        w_ref = rest.pop(0) if w_next is not None else None
        o_ref = rest.pop(0)
        obf_ref = rest.pop(0) if emit_bf16 else None

        r = _bdot(a_ref[...], s_ref[...])
        if w_next is not None:
            r = _bdot(r, w_ref[...])
        if act is not None:
            r = act(r)
        o_ref[...] = r.astype(out_dtype)
        if emit_bf16:
            obf_ref[...] = r.astype(_BF)

    in_specs = [pl.BlockSpec((bm, k), lambda i: (i, 0)),
                pl.BlockSpec((k, f), lambda i: (0, 0))]
    args = [adj, s]
    if w_next is not None:
        in_specs.append(pl.BlockSpec(w_next.shape, lambda i: (0, 0)))
        args.append(w_next)
    out_specs = [pl.BlockSpec((bm, fo), lambda i: (i, 0))]
    out_shape = [jax.ShapeDtypeStruct((m, fo), out_dtype)]
    if emit_bf16:
        out_specs.append(pl.BlockSpec((bm, fo), lambda i: (i, 0)))
        out_shape.append(jax.ShapeDtypeStruct((m, fo), _BF))
    res = pl.pallas_call(
        kern,
        grid=(m // bm,),
        in_specs=in_specs,
        out_specs=out_specs,
        out_shape=out_shape,
        compiler_params=_CP(dimension_semantics=("parallel",)),
    )(*args)
    return res if emit_bf16 else res[0]


def _gae_encoder(adj, s1, w2, w3, fuse_z=None, emit_bf16_adj=False):
    m, k = adj.shape
    bs = 256
    bm = 512
    n1 = m // bs
    n23 = m // bm
    n_stages = 4 if fuse_z is not None else 3
    grid = n1 + (n_stages - 1) * n23

    def kern(a_ref, s1_ref, w2_ref, w3_ref, *rest):
        rest = list(rest)
        if fuse_z is not None:
            pk_ref = rest.pop(0)
        zig_ref = rest.pop(0)
        z_ref = rest.pop(0) if fuse_z is not None else None
        abf_ref = rest.pop(0) if emit_bf16_adj else None
        amv, s2v, s3v, zigv = rest
        i = pl.program_id(0)

        @pl.when(i < n1)
        def _stage1():
            a = a_ref[...].astype(_BF)
            if emit_bf16_adj:
                abf_ref[...] = a
            amv[pl.ds(i * bs, bs), :] = a
            r = _dot(a, s1_ref[...])
            r = jnp.tanh(_bdot(r, w2_ref[...]))
            s2v[pl.ds(i * bs, bs), :] = r.astype(_BF)

        @pl.when((i >= n1) & (i < n1 + n23))
        def _stage2():
            j = i - n1
            a = amv[pl.ds(j * bm, bm), :]
            r = _dot(a, s2v[...])
            s3v[pl.ds(j * bm, bm), :] = _bdot(r, w3_ref[...]).astype(_BF)

        @pl.when((i >= n1 + n23) & (i < n1 + 2 * n23))
        def _stage3():
            j = i - n1 - n23
            a = amv[pl.ds(j * bm, bm), :]
            zg = _dot(a, s3v[...])
            zig_ref[...] = zg
            zigv[pl.ds(j * bm, bm), :] = zg

        if fuse_z is not None:
            @pl.when(i >= n1 + 2 * n23)
            def _stage4():
                j = i - n1 - 2 * n23
                a = amv[pl.ds(j * bm, bm), :]
                pk = pk_ref[...]
                g2, e1, e2 = pk[:, 0:20], pk[:, 20:40], pk[:, 40:60]
                fa, fb = pk[:, 60:80], pk[:, 80:100]
                zi = (fa * 0.5 * (e1 + e2) + fb * 0.5 * (zigv[...] + g2))
                z_ref[...] = _bdot(a, zi)

    def _in_idx(i):
        return (jnp.minimum(i, n1 - 1), 0)

    def _zig_idx(i):
        return (jnp.clip(i - n1 - n23, 0, n23 - 1), 0)

    def _z_idx(i):
        return (jnp.clip(i - n1 - 2 * n23, 0, n23 - 1), 0)

    in_specs = [pl.BlockSpec((bs, k), _in_idx),
                pl.BlockSpec(s1.shape, lambda i: (0, 0)),
                pl.BlockSpec(w2.shape, lambda i: (0, 0)),
                pl.BlockSpec(w3.shape, lambda i: (0, 0))]
    args = [adj, s1, w2, w3]
    if fuse_z is not None:
        packed = jnp.concatenate(list(fuse_z), axis=1)
        in_specs.append(pl.BlockSpec(packed.shape, lambda i: (0, 0)))
        args.append(packed)
    out_specs = [pl.BlockSpec((bm, 20), _zig_idx)]
    out_shape = [jax.ShapeDtypeStruct((m, 20), jnp.float32)]
    if fuse_z is not None:
        out_specs.append(pl.BlockSpec((bm, 20), _z_idx))
        out_shape.append(jax.ShapeDtypeStruct((m, 20), jnp.float32))
    if emit_bf16_adj:
        out_specs.append(pl.BlockSpec((bs, k), _in_idx))
        out_shape.append(jax.ShapeDtypeStruct((m, k), _BF))
    return pl.pallas_call(
        kern,
        grid=(grid,),
        in_specs=in_specs,
        out_specs=out_specs,
        out_shape=out_shape,
        scratch_shapes=[pltpu.VMEM((m, k), _BF),
                        pltpu.VMEM((m, w2.shape[1]), _BF),
                        pltpu.VMEM((m, w3.shape[1]), _BF),
                        pltpu.VMEM((m, 20), jnp.float32)],
        compiler_params=_CP(dimension_semantics=("arbitrary",)),
    )(*args)


def _ae_dec_kernel(am_ref, pk_ref, w1, b1, w2, b2, w3, b3, wx, bx, wm, bm_,
                  wd, bd, wp, bp, wg4,
                  z_ref, xh_ref, mean_ref, disp_ref, pi_ref, s4_ref):
    pk = pk_ref[...]
    g1, g2, e1, e2 = pk[:, 0:20], pk[:, 20:40], pk[:, 40:60], pk[:, 60:80]
    fa, fb = pk[:, 80:100], pk[:, 100:120]
    zi = fa * 0.5 * (e1 + e2) + fb * 0.5 * (g1 + g2)
    z = _bdot(am_ref[...], zi)
    z_ref[...] = z
    h = _leaky(_bdot(z, w1[...]) + b1[...])
    h = _leaky(_bdot(h, w2[...]) + b2[...])
    h = _leaky(_bdot(h, w3[...]) + b3[...])
    xh_ref[...] = _bdot(h, wx[...]) + bx[...]
    mean_ref[...] = jnp.clip(jnp.exp(_bdot(h, wm[...]) + bm_[...]), 1e-5, 1e6)
    disp_ref[...] = jnp.clip(jax.nn.softplus(_bdot(h, wd[...]) + bd[...]),
                             1e-4, 1e4)
    pi_ref[...] = jax.nn.sigmoid(_bdot(h, wp[...]) + bp[...])
    s4_ref[...] = jnp.tanh(_bdot(z, wg4[...])).astype(_BF)


def _ae_dec(am_bf, zig1, zig2, zae1, zae2, p):
    m = am_bf.shape[0]
    bm = 512
    ni = p['ae_xbar_W'].shape[1]
    packed = jnp.concatenate([zig1, zig2, zae1, zae2, p['a'], p['b']],
                             axis=1)
    ws = [p['ae_d1_W'], p['ae_d1_b'].reshape(1, -1),
          p['ae_d2_W'], p['ae_d2_b'].reshape(1, -1),
          p['ae_d3_W'], p['ae_d3_b'].reshape(1, -1),
          p['ae_xbar_W'], p['ae_xbar_b'].reshape(1, -1),
          p['ae_mean_W'], p['ae_mean_b'].reshape(1, -1),
          p['ae_disp_W'], p['ae_disp_b'].reshape(1, -1),
          p['ae_pi_W'], p['ae_pi_b'].reshape(1, -1),
          p['g_d4_W']]
    in_specs = [pl.BlockSpec((bm, m), lambda i: (i, 0)),
                pl.BlockSpec(packed.shape, lambda i: (0, 0))]
    in_specs += [pl.BlockSpec(w.shape, lambda i: (0, 0)) for w in ws]
    big = pl.BlockSpec((bm, ni), lambda i: (i, 0))
    big_s = jax.ShapeDtypeStruct((m, ni), jnp.float32)
    return pl.pallas_call(
        _ae_dec_kernel,
        grid=(m // bm,),
        in_specs=in_specs,
        out_specs=[pl.BlockSpec((bm, 20), lambda i: (i, 0)),
                   big, big, big, big,
                   pl.BlockSpec((bm, 256), lambda i: (i, 0))],
        out_shape=[jax.ShapeDtypeStruct((m, 20), jnp.float32),
                   big_s, big_s, big_s, big_s,
                   jax.ShapeDtypeStruct((m, 256), _BF)],
        compiler_params=_CP(dimension_semantics=("parallel",)),
    )(am_bf, packed, *ws)


def _gae_decoder(am_bf, s4, w5, w6, bm=512):
    m, k = am_bf.shape
    n = m // bm
    ni = w6.shape[1]

    def kern(a_ref, s4_ref, w5_ref, w6_ref, zh_ref, zhbf_ref, s5v, s6v):
        i = pl.program_id(0)

        @pl.when(i < n)
        def _stage1():
            a = a_ref[pl.ds(i * bm, bm), :]
            r = _dot(a, s4_ref[...])
            s5v[pl.ds(i * bm, bm), :] = jnp.tanh(_bdot(r, w5_ref[...])).astype(_BF)

        @pl.when((i >= n) & (i < 2 * n))
        def _stage2():
            j = i - n
            a = a_ref[pl.ds(j * bm, bm), :]
            r = _dot(a, s5v[...])
            s6v[pl.ds(j * bm, bm), :] = jnp.tanh(_bdot(r, w6_ref[...])).astype(_BF)

        @pl.when(i >= 2 * n)
        def _stage3():
            j = i - 2 * n
            a = a_ref[pl.ds(j * bm, bm), :]
            r = _dot(a, s6v[...])
            zh_ref[...] = r
            zhbf_ref[...] = r.astype(_BF)

    def _o_idx(i):
        return (jnp.clip(i - 2 * n, 0, n - 1), 0)

    return pl.pallas_call(
        kern,
        grid=(3 * n,),
        in_specs=[pl.BlockSpec(am_bf.shape, lambda i: (0, 0)),
                  pl.BlockSpec(s4.shape, lambda i: (0, 0)),
                  pl.BlockSpec(w5.shape, lambda i: (0, 0)),
                  pl.BlockSpec(w6.shape, lambda i: (0, 0))],
        out_specs=[pl.BlockSpec((bm, ni), _o_idx),
                   pl.BlockSpec((bm, ni), _o_idx)],
        out_shape=[jax.ShapeDtypeStruct((m, ni), jnp.float32),
                   jax.ShapeDtypeStruct((m, ni), _BF)],
        scratch_shapes=[pltpu.VMEM((m, w5.shape[1]), _BF),
                        pltpu.VMEM((m, ni), _BF)],
        compiler_params=_CP(dimension_semantics=("arbitrary",)),
    )(am_bf, s4, w5, w6)


def _a_hat(zig1, zig2, zh_bf, bm=512):
    m = zig1.shape[0]

    def kern(b1_ref, t1_ref, b2_ref, t2_ref, bh_ref, th_ref, o_ref):
        r = 0.5 * jax.nn.sigmoid(_bdot_t(b1_ref[...], t1_ref[...]))
        r += 0.5 * jax.nn.sigmoid(_bdot_t(b2_ref[...], t2_ref[...]))
        r += jax.nn.sigmoid(_bdot_t(bh_ref[...], th_ref[...]))
        o_ref[...] = r

    blk = pl.BlockSpec((bm, 20), lambda i: (i, 0))
    full = pl.BlockSpec((m, 20), lambda i: (0, 0))
    return pl.pallas_call(
        kern,
        grid=(m // bm,),
        in_specs=[blk, full, blk, full,
                  pl.BlockSpec((bm, zh_bf.shape[1]), lambda i: (i, 0)),
                  pl.BlockSpec(zh_bf.shape, lambda i: (0, 0))],
        out_specs=pl.BlockSpec((bm, m), lambda i: (i, 0)),
        out_shape=jax.ShapeDtypeStruct((m, m), jnp.float32),
        compiler_params=_CP(dimension_semantics=("parallel",)),
    )(zig1, zig1, zig2, zig2, zh_bf, zh_bf)


def kernel(X_tilde1, Am, X_tilde2, Ad, params):
    p = params
    zae1, zae2, s1a, s1b = _enc0(X_tilde1, X_tilde2, p)

    (zig2,) = _gae_encoder(Ad, s1b, p['g_e2_W'], p['g_e3_W'])
    zig1, am_bf = _gae_encoder(Am, s1a, p['g_e2_W'], p['g_e3_W'],
                               emit_bf16_adj=True)

    z, x_hat, mean, disp, pi, s4 = _ae_dec(am_bf, zig1, zig2, zae1, zae2, p)

    z_hat, zh_bf = _gae_decoder(am_bf, s4, p['g_d5_W'], p['g_d6_W'])

    a_hat = _a_hat(zig1, zig2, zh_bf)
    return x_hat, mean, disp, pi, z_hat, a_hat, z


# speedup vs baseline: 1.0249x; 1.0249x over previous
import jax
import jax.numpy as jnp
from jax.experimental import pallas as pl
from jax.experimental.pallas import tpu as pltpu

_N = 4096
_BF = jnp.bfloat16

_CP = getattr(pltpu, "CompilerParams", None) or getattr(pltpu, "TPUCompilerParams")


def _leaky(x):
    return jnp.where(x > 0, x, 0.2 * x)


def _dot(a, b):
    return jnp.dot(a, b, preferred_element_type=jnp.float32)


def _bdot(a, b):
    return jnp.dot(a.astype(_BF), b.astype(_BF), preferred_element_type=jnp.float32)


def _bdot_t(a, b, out_dtype=jnp.float32):
    return jax.lax.dot_general(
        a.astype(_BF), b.astype(_BF),
        dimension_numbers=(((1,), (1,)), ((), ())),
        preferred_element_type=out_dtype)


def _enc0_kernel(x1_ref, x2_ref, w1, b1, w2, b2, w3, b3, wz, bz, g1,
                 zae1_ref, zae2_ref, s1a_ref, s1b_ref):
    for x_ref, zae_ref, s1_ref in ((x1_ref, zae1_ref, s1a_ref),
                                   (x2_ref, zae2_ref, s1b_ref)):
        x = x_ref[...]
        h = _leaky(_bdot(x, w1[...]) + b1[...])
        h = _leaky(_bdot(h, w2[...]) + b2[...])
        h = _leaky(_bdot(h, w3[...]) + b3[...])
        zae_ref[...] = _dot(h, wz[...]) + bz[...]
        s1_ref[...] = jnp.tanh(_bdot(x, g1[...])).astype(_BF)


def _enc0(x1, x2, p):
    m = x1.shape[0]
    bm = 512
    ws = [p['ae_e1_W'], p['ae_e1_b'].reshape(1, -1),
          p['ae_e2_W'], p['ae_e2_b'].reshape(1, -1),
          p['ae_e3_W'], p['ae_e3_b'].reshape(1, -1),
          p['ae_z_W'], p['ae_z_b'].reshape(1, -1),
          p['g_e1_W']]
    xspec = pl.BlockSpec((bm, x1.shape[1]), lambda i: (i, 0))
    in_specs = [xspec, xspec]
    in_specs += [pl.BlockSpec(w.shape, lambda i: (0, 0)) for w in ws]
    lat = pl.BlockSpec((bm, 20), lambda i: (i, 0))
    s1 = pl.BlockSpec((bm, 128), lambda i: (i, 0))
    return pl.pallas_call(
        _enc0_kernel,
        grid=(m // bm,),
        in_specs=in_specs,
        out_specs=[lat, lat, s1, s1],
        out_shape=[jax.ShapeDtypeStruct((m, 20), jnp.float32),
                   jax.ShapeDtypeStruct((m, 20), jnp.float32),
                   jax.ShapeDtypeStruct((m, 128), _BF),
                   jax.ShapeDtypeStruct((m, 128), _BF)],
        compiler_params=_CP(dimension_semantics=("parallel",)),
    )(x1, x2, *ws)


def _gae_encoder(adj, s1, w2, w3, emit_bf16_adj=False):
    m, k = adj.shape
    bs = 256
    bm = 512
    n1 = m // bs
    n23 = m // bm
    grid = n1 + 2 * n23

    def kern(a_ref, s1_ref, w2_ref, w3_ref, *rest):
        rest = list(rest)
        zig_ref = rest.pop(0)
        abf_ref = rest.pop(0) if emit_bf16_adj else None
        amv, s2v, s3v = rest
        i = pl.program_id(0)

        @pl.when(i < n1)
        def _stage1():
            a = a_ref[...].astype(_BF)
            if emit_bf16_adj:
                abf_ref[...] = a
            amv[pl.ds(i * bs, bs), :] = a
            r = _dot(a, s1_ref[...])
            r = jnp.tanh(_bdot(r, w2_ref[...]))
            s2v[pl.ds(i * bs, bs), :] = r.astype(_BF)

        @pl.when((i >= n1) & (i < n1 + n23))
        def _stage2():
            j = i - n1
            a = amv[pl.ds(j * bm, bm), :]
            r = _dot(a, s2v[...])
            s3v[pl.ds(j * bm, bm), :] = _bdot(r, w3_ref[...]).astype(_BF)

        @pl.when(i >= n1 + n23)
        def _stage3():
            j = i - n1 - n23
            a = amv[pl.ds(j * bm, bm), :]
            zig_ref[...] = _dot(a, s3v[...])

    def _in_idx(i):
        return (jnp.minimum(i, n1 - 1), 0)

    def _zig_idx(i):
        return (jnp.clip(i - n1 - n23, 0, n23 - 1), 0)

    in_specs = [pl.BlockSpec((bs, k), _in_idx),
                pl.BlockSpec(s1.shape, lambda i: (0, 0)),
                pl.BlockSpec(w2.shape, lambda i: (0, 0)),
                pl.BlockSpec(w3.shape, lambda i: (0, 0))]
    out_specs = [pl.BlockSpec((bm, 20), _zig_idx)]
    out_shape = [jax.ShapeDtypeStruct((m, 20), jnp.float32)]
    if emit_bf16_adj:
        out_specs.append(pl.BlockSpec((bs, k), _in_idx))
        out_shape.append(jax.ShapeDtypeStruct((m, k), _BF))
    return pl.pallas_call(
        kern,
        grid=(grid,),
        in_specs=in_specs,
        out_specs=out_specs,
        out_shape=out_shape,
        scratch_shapes=[pltpu.VMEM((m, k), _BF),
                        pltpu.VMEM((m, w2.shape[1]), _BF),
                        pltpu.VMEM((m, w3.shape[1]), _BF)],
        compiler_params=_CP(dimension_semantics=("arbitrary",)),
    )(adj, s1, w2, w3)


def _ae_dec_kernel(am_ref, a_ref, b_ref, e1_ref, e2_ref, g1_ref, g2_ref,
                  w1, b1, w2, b2, w3, b3, wx, bx, wm, bm_,
                  wd, bd, wp, bp, wg4,
                  z_ref, xh_ref, mean_ref, disp_ref, pi_ref, s4_ref):
    zi = (a_ref[...] * 0.5 * (e1_ref[...] + e2_ref[...])
          + b_ref[...] * 0.5 * (g1_ref[...] + g2_ref[...]))
    z = _bdot(am_ref[...], zi)
    z_ref[...] = z
    h = _leaky(_bdot(z, w1[...]) + b1[...])
    h = _leaky(_bdot(h, w2[...]) + b2[...])
    h = _leaky(_bdot(h, w3[...]) + b3[...])
    xh_ref[...] = _bdot(h, wx[...]) + bx[...]
    mean_ref[...] = jnp.clip(jnp.exp(_bdot(h, wm[...]) + bm_[...]), 1e-5, 1e6)
    disp_ref[...] = jnp.clip(jax.nn.softplus(_bdot(h, wd[...]) + bd[...]),
                             1e-4, 1e4)
    pi_ref[...] = jax.nn.sigmoid(_bdot(h, wp[...]) + bp[...])
    s4_ref[...] = jnp.tanh(_bdot(z, wg4[...])).astype(_BF)


def _ae_dec(am_bf, a, b, zae1, zae2, zig1, zig2, p):
    m = am_bf.shape[0]
    bm = 512
    ni = p['ae_xbar_W'].shape[1]
    ws = [p['ae_d1_W'], p['ae_d1_b'].reshape(1, -1),
          p['ae_d2_W'], p['ae_d2_b'].reshape(1, -1),
          p['ae_d3_W'], p['ae_d3_b'].reshape(1, -1),
          p['ae_xbar_W'], p['ae_xbar_b'].reshape(1, -1),
          p['ae_mean_W'], p['ae_mean_b'].reshape(1, -1),
          p['ae_disp_W'], p['ae_disp_b'].reshape(1, -1),
          p['ae_pi_W'], p['ae_pi_b'].reshape(1, -1),
          p['g_d4_W']]
    col = pl.BlockSpec((m, 20), lambda i: (0, 0))
    in_specs = [pl.BlockSpec((bm, m), lambda i: (i, 0)),
                col, col, col, col, col, col]
    in_specs += [pl.BlockSpec(w.shape, lambda i: (0, 0)) for w in ws]
    big = pl.BlockSpec((bm, ni), lambda i: (i, 0))
    big_s = jax.ShapeDtypeStruct((m, ni), jnp.float32)
    return pl.pallas_call(
        _ae_dec_kernel,
        grid=(m // bm,),
        in_specs=in_specs,
        out_specs=[pl.BlockSpec((bm, 20), lambda i: (i, 0)),
                   big, big, big, big,
                   pl.BlockSpec((bm, 256), lambda i: (i, 0))],
        out_shape=[jax.ShapeDtypeStruct((m, 20), jnp.float32),
                   big_s, big_s, big_s, big_s,
                   jax.ShapeDtypeStruct((m, 256), _BF)],
        compiler_params=_CP(dimension_semantics=("parallel",)),
    )(am_bf, a, b, zae1, zae2, zig1, zig2, *ws)


def _gae_decoder(am_bf, s4, w5, w6, bm=512):
    m, k = am_bf.shape
    n = m // bm
    ni = w6.shape[1]

    def kern(a_ref, s4_ref, w5_ref, w6_ref, zh_ref, zhbf_ref, s5v, s6v):
        i = pl.program_id(0)

        @pl.when(i < n)
        def _stage1():
            a = a_ref[pl.ds(i * bm, bm), :]
            r = _dot(a, s4_ref[...])
            s5v[pl.ds(i * bm, bm), :] = jnp.tanh(_bdot(r, w5_ref[...])).astype(_BF)

        @pl.when((i >= n) & (i < 2 * n))
        def _stage2():
            j = i - n
            a = a_ref[pl.ds(j * bm, bm), :]
            r = _dot(a, s5v[...])
            s6v[pl.ds(j * bm, bm), :] = jnp.tanh(_bdot(r, w6_ref[...])).astype(_BF)

        @pl.when(i >= 2 * n)
        def _stage3():
            j = i - 2 * n
            a = a_ref[pl.ds(j * bm, bm), :]
            r = _dot(a, s6v[...])
            zh_ref[...] = r
            zhbf_ref[...] = r.astype(_BF)

    def _o_idx(i):
        return (jnp.clip(i - 2 * n, 0, n - 1), 0)

    return pl.pallas_call(
        kern,
        grid=(3 * n,),
        in_specs=[pl.BlockSpec(am_bf.shape, lambda i: (0, 0)),
                  pl.BlockSpec(s4.shape, lambda i: (0, 0)),
                  pl.BlockSpec(w5.shape, lambda i: (0, 0)),
                  pl.BlockSpec(w6.shape, lambda i: (0, 0))],
        out_specs=[pl.BlockSpec((bm, ni), _o_idx),
                   pl.BlockSpec((bm, ni), _o_idx)],
        out_shape=[jax.ShapeDtypeStruct((m, ni), jnp.float32),
                   jax.ShapeDtypeStruct((m, ni), _BF)],
        scratch_shapes=[pltpu.VMEM((m, w5.shape[1]), _BF),
                        pltpu.VMEM((m, ni), _BF)],
        compiler_params=_CP(dimension_semantics=("arbitrary",)),
    )(am_bf, s4, w5, w6)


def _a_hat(zig1, zig2, zh_bf, bm=512):
    m = zig1.shape[0]

    def kern(b1_ref, t1_ref, b2_ref, t2_ref, bh_ref, th_ref, o_ref):
        r = 0.5 * jax.nn.sigmoid(_bdot_t(b1_ref[...], t1_ref[...]))
        r += 0.5 * jax.nn.sigmoid(_bdot_t(b2_ref[...], t2_ref[...]))
        r += jax.nn.sigmoid(_bdot_t(bh_ref[...], th_ref[...]))
        o_ref[...] = r

    blk = pl.BlockSpec((bm, 20), lambda i: (i, 0))
    full = pl.BlockSpec((m, 20), lambda i: (0, 0))
    return pl.pallas_call(
        kern,
        grid=(m // bm,),
        in_specs=[blk, full, blk, full,
                  pl.BlockSpec((bm, zh_bf.shape[1]), lambda i: (i, 0)),
                  pl.BlockSpec(zh_bf.shape, lambda i: (0, 0))],
        out_specs=pl.BlockSpec((bm, m), lambda i: (i, 0)),
        out_shape=jax.ShapeDtypeStruct((m, m), jnp.float32),
        compiler_params=_CP(dimension_semantics=("parallel",)),
    )(zig1, zig1, zig2, zig2, zh_bf, zh_bf)


def kernel(X_tilde1, Am, X_tilde2, Ad, params):
    p = params
    zae1, zae2, s1a, s1b = _enc0(X_tilde1, X_tilde2, p)

    (zig2,) = _gae_encoder(Ad, s1b, p['g_e2_W'], p['g_e3_W'])
    zig1, am_bf = _gae_encoder(Am, s1a, p['g_e2_W'], p['g_e3_W'],
                               emit_bf16_adj=True)

    z, x_hat, mean, disp, pi, s4 = _ae_dec(am_bf, p['a'], p['b'],
                                           zae1, zae2, zig1, zig2, p)

    z_hat, zh_bf = _gae_decoder(am_bf, s4, p['g_d5_W'], p['g_d6_W'])

    a_hat = _a_hat(zig1, zig2, zh_bf)
    return x_hat, mean, disp, pi, z_hat, a_hat, z
